# baseline (speedup 1.0000x reference)
import functools

import jax
import jax.numpy as jnp
from jax import lax
from jax.experimental import pallas as pl
from jax.experimental.pallas import tpu as pltpu
from jax.experimental.pallas import tpu_sc as plsc

NUM_NODES = 100000
MEM_DIM = 128
MSG_DIM = 128
TIME_DIM = 64
NUM_EVENTS = 200000
B = 4096
N_SUB = 16384
E_SUB = 81920

_NC = 2
_NS = 16
_NW = _NC * _NS

@functools.cache
def _mesh():
    return plsc.VectorSubcoreMesh(core_axis_name="c", subcore_axis_name="s",
                                  num_cores=_NC, num_subcores=_NS)

_HALF = NUM_NODES // 2
_ACC_OWN = N_SUB // 2
_ACC_TRASH = 512
_ACC_ROWS = _ACC_OWN + _ACC_TRASH


def _sc_stage1_body(mem_hbm, lu_hbm, ft_hbm, nid_hbm, ssub_hbm, eid_hbm,
                    fmsg_hbm, srcdst_hbm,
                    h_out, msd_out, relt_out, hfeat_out, msgf_out,
                    nid_v, idx_v, node_v, rows_v, idx256_v, rows256_v,
                    fa_v, fb_v, sem):
    c = lax.axis_index("c")
    s = lax.axis_index("s")
    w = c * _NS + s

    hb = w * (N_SUB // _NW)
    pltpu.sync_copy(nid_hbm.at[pl.ds(hb, 512)], idx_v)
    pltpu.async_copy(mem_hbm.at[idx_v], rows_v, sem).wait()
    pltpu.sync_copy(rows_v, h_out.at[pl.ds(hb, 512)])
    del nid_v

    ub = w * (2 * B // _NW)
    pltpu.sync_copy(srcdst_hbm.at[pl.ds(ub, 256)], idx256_v)
    pltpu.async_copy(mem_hbm.at[idx256_v], rows256_v, sem).wait()
    pltpu.sync_copy(rows256_v, msd_out.at[pl.ds(ub, 256)])

    for k in range(E_SUB // _NW // 512):
        eb = w * (E_SUB // _NW) + k * 512
        pltpu.sync_copy(ssub_hbm.at[pl.ds(eb, 512)], idx_v)
        pltpu.async_copy(nid_hbm.at[idx_v], node_v, sem).wait()
        pltpu.async_copy(lu_hbm.at[node_v], fa_v, sem).wait()
        pltpu.async_copy(mem_hbm.at[node_v], rows_v, sem).wait()
        pltpu.sync_copy(rows_v, hfeat_out.at[pl.ds(eb, 512)])
        pltpu.sync_copy(eid_hbm.at[pl.ds(eb, 512)], idx_v)
        pltpu.async_copy(ft_hbm.at[idx_v], fb_v, sem).wait()
        pltpu.async_copy(fmsg_hbm.at[idx_v], rows_v, sem).wait()
        pltpu.sync_copy(rows_v, msgf_out.at[pl.ds(eb, 512)])

        def _s(j, _):
            fa_v[pl.ds(j * 16, 16)] = (fa_v[pl.ds(j * 16, 16)]
                                       - fb_v[pl.ds(j * 16, 16)])
            return 0

        lax.fori_loop(0, 32, _s, 0)
        pltpu.sync_copy(fa_v, relt_out.at[pl.ds(eb, 512)])


@functools.cache
def _sc_stage1():
  return pl.kernel(
    _sc_stage1_body,
    out_type=[
        jax.ShapeDtypeStruct((N_SUB, MEM_DIM), jnp.float32),
        jax.ShapeDtypeStruct((2 * B, MEM_DIM), jnp.float32),
        jax.ShapeDtypeStruct((E_SUB,), jnp.float32),
        jax.ShapeDtypeStruct((E_SUB, MEM_DIM), jnp.float32),
        jax.ShapeDtypeStruct((E_SUB, MSG_DIM), jnp.float32),
    ],
    mesh=_mesh(),
    scratch_types=[
        pltpu.VMEM((16,), jnp.int32),
        pltpu.VMEM((512,), jnp.int32),
        pltpu.VMEM((512,), jnp.int32),
        pltpu.VMEM((512, MEM_DIM), jnp.float32),
        pltpu.VMEM((256,), jnp.int32),
        pltpu.VMEM((256, MEM_DIM), jnp.float32),
        pltpu.VMEM((512,), jnp.float32),
        pltpu.VMEM((512,), jnp.float32),
        pltpu.SemaphoreType.DMA,
    ],
  )


def _sc_segsum_body(m_hbm, dsub_hbm,
                    agg_out,
                    dst_v, rows_v, acc, sem):
    c = lax.axis_index("c")
    s = lax.axis_index("s")
    del sem
    coff = c * _ACC_OWN

    def _z(i, _):
        rows_v[i // 8, pl.ds((i % 8) * 16, 16)] = jnp.zeros((16,), jnp.float32)
        return 0

    lax.fori_loop(0, 256 * 8, _z, 0)
    pltpu.sync_copy(rows_v, acc.at[pl.ds(s * 544, 256)])
    pltpu.sync_copy(rows_v, acc.at[pl.ds(s * 544 + 256, 256)])
    pltpu.sync_copy(rows_v.at[pl.ds(0, 32)], acc.at[pl.ds(s * 544 + 512, 32)])
    plsc.subcore_barrier()

    for k in range(20):
        eb = s * (E_SUB // _NS) + k * 256
        pltpu.sync_copy(dsub_hbm.at[pl.ds(eb, 256)], dst_v)
        pltpu.sync_copy(m_hbm.at[pl.ds(eb, 256)], rows_v)

        def _rm(j, _):
            d16 = dst_v[pl.ds(j * 16, 16)]
            loc = d16 - coff
            owned = (loc >= 0) & (loc < _ACC_OWN)
            trash = _ACC_OWN + ((lax.iota(jnp.int32, 16) + j * 16) & 511)
            dst_v[pl.ds(j * 16, 16)] = jnp.where(owned, loc, trash)
            return 0

        lax.fori_loop(0, 16, _rm, 0)
        pltpu.sync_copy(rows_v, acc.at[dst_v], add=True)
    plsc.subcore_barrier()
    pltpu.sync_copy(acc.at[pl.ds(s * 512, 512)],
                    agg_out.at[pl.ds(coff + s * 512, 512)])


@functools.cache
def _sc_segsum():
  return pl.kernel(
    _sc_segsum_body,
    out_type=[
        jax.ShapeDtypeStruct((N_SUB, MEM_DIM), jnp.float32),
    ],
    mesh=_mesh(),
    scratch_types=[
        pltpu.VMEM((256,), jnp.int32),
        pltpu.VMEM((256, MEM_DIM), jnp.float32),
        pltpu.VMEM_SHARED((_ACC_ROWS, MEM_DIM), jnp.float32),
        pltpu.SemaphoreType.DMA,
    ],
  )


def _sc_asm_body(mem_hbm, lu_hbm, hout_hbm, upd_hbm, pgidx_hbm, prow_hbm,
                 pval_hbm, hidx_hbm,
                 nmem_out, nlu_out, hsd_out,
                 idx256_v, rows256_v, idx_v, rowi_v, rows_v, fa_v, sem, sem2):
    c = lax.axis_index("c")
    s = lax.axis_index("s")
    w = c * _NS + s

    pltpu.sync_copy(hidx_hbm.at[pl.ds(w * 256, 256)], idx256_v)
    pltpu.async_copy(hout_hbm.at[idx256_v], rows256_v, sem).wait()
    pltpu.sync_copy(rows256_v, hsd_out.at[pl.ds(w * 256, 256)])

    mb = c * _HALF + s * 3128

    def _copy_rows(off, n, buf):
        pltpu.sync_copy(mem_hbm.at[pl.ds(off, n)], buf)
        pltpu.sync_copy(buf, nmem_out.at[pl.ds(off, n)])

    for k in range(6):
        _copy_rows(mb + k * 512, 512, rows_v)
    _copy_rows(mb + 3072, 8, rows_v.at[pl.ds(0, 8)])
    pltpu.sync_copy(lu_hbm.at[pl.ds(mb, 512)], fa_v)
    pltpu.sync_copy(fa_v, nlu_out.at[pl.ds(mb, 512)])
    for k in range(1, 6):
        pltpu.sync_copy(lu_hbm.at[pl.ds(mb + k * 512, 512)], fa_v)
        pltpu.sync_copy(fa_v, nlu_out.at[pl.ds(mb + k * 512, 512)])
    pltpu.sync_copy(lu_hbm.at[pl.ds(mb + 3072, 8)], fa_v.at[pl.ds(0, 8)])
    pltpu.sync_copy(fa_v.at[pl.ds(0, 8)], nlu_out.at[pl.ds(mb + 3072, 8)])

    @pl.when(s < _NS - 1)
    def _():
        _copy_rows(mb + 3080, 48, rows_v.at[pl.ds(0, 48)])
        pltpu.sync_copy(lu_hbm.at[pl.ds(mb + 3080, 48)],
                        fa_v.at[pl.ds(0, 48)])
        pltpu.sync_copy(fa_v.at[pl.ds(0, 48)],
                        nlu_out.at[pl.ds(mb + 3080, 48)])

    plsc.subcore_barrier()

    pb = c * 2 * B + s * 512
    pltpu.sync_copy(pgidx_hbm.at[pl.ds(pb, 512)], idx_v)
    pltpu.async_copy(upd_hbm.at[idx_v], rows_v, sem).wait()
    pltpu.sync_copy(prow_hbm.at[pl.ds(pb, 512)], rowi_v)
    pltpu.sync_copy(pval_hbm.at[pl.ds(pb, 512)], fa_v)
    d1 = pltpu.async_copy(rows_v, nmem_out.at[rowi_v], sem)
    d2 = pltpu.async_copy(fa_v, nlu_out.at[rowi_v], sem2)
    d1.wait()
    d2.wait()


@functools.cache
def _sc_asm():
  return pl.kernel(
    _sc_asm_body,
    out_type=[
        jax.ShapeDtypeStruct((NUM_NODES, MEM_DIM), jnp.float32),
        jax.ShapeDtypeStruct((NUM_NODES,), jnp.float32),
        jax.ShapeDtypeStruct((2 * B, MEM_DIM), jnp.float32),
    ],
    mesh=_mesh(),
    scratch_types=[
        pltpu.VMEM((256,), jnp.int32),
        pltpu.VMEM((256, MEM_DIM), jnp.float32),
        pltpu.VMEM((512,), jnp.int32),
        pltpu.VMEM((512,), jnp.int32),
        pltpu.VMEM((512, MEM_DIM), jnp.float32),
        pltpu.VMEM((512,), jnp.float32),
        pltpu.SemaphoreType.DMA,
        pltpu.SemaphoreType.DMA,
    ],
  )


_BE = 8192


def _edge_msg_body(hfeat_ref, msgf_ref, relt_ref, wn_h_ref, wn_m_ref,
                   wn_t_ref, tw_ref, tb_ref, out_ref):
    f32 = jnp.float32
    enc = jnp.cos(relt_ref[...][:, None] * tw_ref[...][None, :]
                  + tb_ref[...][None, :])
    m = (
        jnp.dot(hfeat_ref[...], wn_h_ref[...], preferred_element_type=f32)
        + jnp.dot(msgf_ref[...], wn_m_ref[...], preferred_element_type=f32)
        + jnp.dot(enc, wn_t_ref[...], preferred_element_type=f32)
    )
    out_ref[...] = m


def _encoder_body(h_ref, agg_ref, wself_ref, b_ref, out_ref):
    f32 = jnp.float32
    hw = jnp.dot(h_ref[...], wself_ref[...], preferred_element_type=f32)
    out_ref[...] = jnp.tanh(hw + agg_ref[...] + b_ref[...][None, :])


def _update_body(mem_src_ref, mem_dst_ref, msg_ref, wu_mem_ref, wu_msg_ref,
                 bu_ref, out_src_ref, out_dst_ref):
    f32 = jnp.float32
    msg_part = jnp.dot(msg_ref[...], wu_msg_ref[...],
                       preferred_element_type=f32)
    out_src_ref[...] = jnp.tanh(
        jnp.dot(mem_src_ref[...], wu_mem_ref[...], preferred_element_type=f32)
        + msg_part + bu_ref[...][None, :])
    out_dst_ref[...] = jnp.tanh(
        jnp.dot(mem_dst_ref[...], wu_mem_ref[...], preferred_element_type=f32)
        + msg_part + bu_ref[...][None, :])


def kernel(edge_index, t, msg, full_msg, full_t, n_id, sub_edge_index, e_id,
           memory, last_update, time_w, time_b, W_self, W_nbr, b_enc, Wu, bu):
    i32 = jnp.int32
    src = edge_index[0].astype(i32)
    dst = edge_index[1].astype(i32)
    src_sub = sub_edge_index[0].astype(i32)
    dst_sub = sub_edge_index[1].astype(i32)
    e_id = e_id.astype(i32)
    n_id = n_id.astype(i32)

    srcdst = jnp.concatenate([src, dst])
    h, mem_sd, rel_t, h_feats, msg_feats = _sc_stage1()(
        memory, last_update, full_t, n_id, src_sub, e_id, full_msg, srcdst)
    mem_src = mem_sd[:B]
    mem_dst = mem_sd[B:]

    wn_h = W_nbr[:MEM_DIM]
    wn_m = W_nbr[MEM_DIM:MEM_DIM + MSG_DIM]
    wn_t = W_nbr[MEM_DIM + MSG_DIM:]
    m = pl.pallas_call(
        _edge_msg_body,
        grid=(E_SUB // _BE,),
        in_specs=[
            pl.BlockSpec((_BE, MEM_DIM), lambda i: (i, 0)),
            pl.BlockSpec((_BE, MSG_DIM), lambda i: (i, 0)),
            pl.BlockSpec((_BE,), lambda i: (i,)),
            pl.BlockSpec((MEM_DIM, MEM_DIM), lambda i: (0, 0)),
            pl.BlockSpec((MSG_DIM, MEM_DIM), lambda i: (0, 0)),
            pl.BlockSpec((TIME_DIM, MEM_DIM), lambda i: (0, 0)),
            pl.BlockSpec((TIME_DIM,), lambda i: (0,)),
            pl.BlockSpec((TIME_DIM,), lambda i: (0,)),
        ],
        out_specs=pl.BlockSpec((_BE, MEM_DIM), lambda i: (i, 0)),
        out_shape=jax.ShapeDtypeStruct((E_SUB, MEM_DIM), jnp.float32),
    )(h_feats, msg_feats, rel_t, wn_h, wn_m, wn_t, time_w, time_b)

    wu_mem = Wu[:MEM_DIM]
    wu_msg = Wu[MEM_DIM:]
    upd_src, upd_dst = pl.pallas_call(
        _update_body,
        grid=(1,),
        in_specs=[
            pl.BlockSpec((B, MEM_DIM), lambda i: (0, 0)),
            pl.BlockSpec((B, MEM_DIM), lambda i: (0, 0)),
            pl.BlockSpec((B, MSG_DIM), lambda i: (0, 0)),
            pl.BlockSpec((MEM_DIM, MEM_DIM), lambda i: (0, 0)),
            pl.BlockSpec((MSG_DIM, MEM_DIM), lambda i: (0, 0)),
            pl.BlockSpec((MEM_DIM,), lambda i: (0,)),
        ],
        out_specs=[
            pl.BlockSpec((B, MEM_DIM), lambda i: (0, 0)),
            pl.BlockSpec((B, MEM_DIM), lambda i: (0, 0)),
        ],
        out_shape=[
            jax.ShapeDtypeStruct((B, MEM_DIM), jnp.float32),
            jax.ShapeDtypeStruct((B, MEM_DIM), jnp.float32),
        ],
    )(mem_src, mem_dst, msg, wu_mem, wu_msg, bu)

    (agg,) = _sc_segsum()(m, dst_sub)

    BN = 4096
    h_out = pl.pallas_call(
        _encoder_body,
        grid=(N_SUB // BN,),
        in_specs=[
            pl.BlockSpec((BN, MEM_DIM), lambda i: (i, 0)),
            pl.BlockSpec((BN, MEM_DIM), lambda i: (i, 0)),
            pl.BlockSpec((MEM_DIM, MEM_DIM), lambda i: (0, 0)),
            pl.BlockSpec((MEM_DIM,), lambda i: (0,)),
        ],
        out_specs=pl.BlockSpec((BN, MEM_DIM), lambda i: (i, 0)),
        out_shape=jax.ShapeDtypeStruct((N_SUB, MEM_DIM), jnp.float32),
    )(h, agg, W_self, b_enc)

    q = jnp.concatenate([src, dst])
    pos = jnp.searchsorted(n_id, q, side="right").astype(i32) - 1
    found = (pos >= 0) & (n_id[jnp.clip(pos, 0)] == q)
    miss_row = N_SUB + (jnp.arange(2 * B, dtype=i32) % 32)
    hidx = jnp.where(found, pos, miss_row).astype(i32)

    rows_all = q
    ordinal = jnp.arange(2 * B, dtype=i32)
    win_slot = jnp.full((NUM_NODES,), -1, i32).at[rows_all].max(ordinal)
    gidx_all = win_slot[rows_all]
    tval_all = jnp.concatenate([t, t])

    sent_rows = jnp.concatenate([jnp.arange(64, dtype=i32),
                                 jnp.arange(64, dtype=i32) + _HALF])
    sw = win_slot[sent_rows]
    upd_cat = jnp.concatenate([upd_src, upd_dst], axis=0)
    sent_vals = jnp.where(sw[:, None] >= 0, upd_cat[jnp.clip(sw, 0)],
                          memory[sent_rows])
    sent_lu = jnp.where(sw >= 0, tval_all[jnp.clip(sw, 0)],
                        last_update[sent_rows])
    upd_all = jnp.concatenate([upd_cat, sent_vals], axis=0)

    slot_mod = jnp.arange(2 * B, dtype=i32) % 64
    pg, prw, pvl = [], [], []
    for ci in range(2):
        mask = (rows_all >= _HALF) == (ci == 1)
        sel = jnp.where(mask, size=2 * B, fill_value=-1)[0].astype(i32)
        valid = sel >= 0
        selc = jnp.clip(sel, 0)
        pad_g = 2 * B + ci * 64 + slot_mod
        pad_r = ci * _HALF + slot_mod
        pad_v = sent_lu[ci * 64 + slot_mod]
        pg_c = jnp.where(valid, gidx_all[selc], pad_g)
        prw_c = jnp.where(valid, rows_all[selc], pad_r)
        pvl_c = jnp.where(valid, tval_all[gidx_all[selc]], pad_v)
        order = jnp.argsort(prw_c)
        pg.append(pg_c[order])
        prw.append(prw_c[order])
        pvl.append(pvl_c[order])
    pgidx = jnp.concatenate(pg)
    prow = jnp.concatenate(prw)
    pval = jnp.concatenate(pvl)

    h_ext = jnp.concatenate(
        [h_out, jnp.broadcast_to(h_out[0:1], (32, MEM_DIM))], axis=0)
    new_memory, new_last_update, hsd = _sc_asm()(
        memory, last_update, h_ext, upd_all, pgidx, prow, pval, hidx)
    h_src = hsd[:B]
    h_dst = hsd[B:]

    return (h_src, h_dst, new_memory, new_last_update)


# speedup vs baseline: 2.1460x; 2.1460x over previous
import functools

import jax
import jax.numpy as jnp
from jax import lax
from jax.experimental import pallas as pl
from jax.experimental.pallas import tpu as pltpu
from jax.experimental.pallas import tpu_sc as plsc

NUM_NODES = 100000
MEM_DIM = 128
MSG_DIM = 128
TIME_DIM = 64
NUM_EVENTS = 200000
B = 4096
N_SUB = 16384
E_SUB = 81920

_NC = 2
_NS = 16
_NW = _NC * _NS

@functools.cache
def _mesh():
    return plsc.VectorSubcoreMesh(core_axis_name="c", subcore_axis_name="s",
                                  num_cores=_NC, num_subcores=_NS)

_HALF = NUM_NODES // 2
_ACC_OWN = N_SUB // 2
_ACC_TRASH = 512
_ACC_ROWS = _ACC_OWN + _ACC_TRASH


def _sc_stage1_body(mem_hbm, lu_hbm, ft_hbm, nid_hbm, ssub_hbm, eid_hbm,
                    fmsg_hbm, srcdst_hbm,
                    h_out, msd_out, relt_out, hfeat_out, msgf_out,
                    nid_v, idx_v, node_v, rows_v, idx256_v, rows256_v,
                    fa_v, fb_v, sem):
    c = lax.axis_index("c")
    s = lax.axis_index("s")
    w = c * _NS + s

    hb = w * (N_SUB // _NW)
    pltpu.sync_copy(nid_hbm.at[pl.ds(hb, 512)], idx_v)
    pltpu.async_copy(mem_hbm.at[idx_v], rows_v, sem).wait()
    pltpu.sync_copy(rows_v, h_out.at[pl.ds(hb, 512)])
    del nid_v

    ub = w * (2 * B // _NW)
    pltpu.sync_copy(srcdst_hbm.at[pl.ds(ub, 256)], idx256_v)
    pltpu.async_copy(mem_hbm.at[idx256_v], rows256_v, sem).wait()
    pltpu.sync_copy(rows256_v, msd_out.at[pl.ds(ub, 256)])

    for k in range(E_SUB // _NW // 512):
        eb = w * (E_SUB // _NW) + k * 512
        pltpu.sync_copy(ssub_hbm.at[pl.ds(eb, 512)], idx_v)
        pltpu.async_copy(nid_hbm.at[idx_v], node_v, sem).wait()
        pltpu.async_copy(lu_hbm.at[node_v], fa_v, sem).wait()
        pltpu.async_copy(mem_hbm.at[node_v], rows_v, sem).wait()
        pltpu.sync_copy(rows_v, hfeat_out.at[pl.ds(eb, 512)])
        pltpu.sync_copy(eid_hbm.at[pl.ds(eb, 512)], idx_v)
        pltpu.async_copy(ft_hbm.at[idx_v], fb_v, sem).wait()
        pltpu.async_copy(fmsg_hbm.at[idx_v], rows_v, sem).wait()
        pltpu.sync_copy(rows_v, msgf_out.at[pl.ds(eb, 512)])

        def _s(j, _):
            fa_v[pl.ds(j * 16, 16)] = (fa_v[pl.ds(j * 16, 16)]
                                       - fb_v[pl.ds(j * 16, 16)])
            return 0

        lax.fori_loop(0, 32, _s, 0)
        pltpu.sync_copy(fa_v, relt_out.at[pl.ds(eb, 512)])


@functools.cache
def _sc_stage1():
  return pl.kernel(
    _sc_stage1_body,
    out_type=[
        jax.ShapeDtypeStruct((N_SUB, MEM_DIM), jnp.float32),
        jax.ShapeDtypeStruct((2 * B, MEM_DIM), jnp.float32),
        jax.ShapeDtypeStruct((E_SUB,), jnp.float32),
        jax.ShapeDtypeStruct((E_SUB, MEM_DIM), jnp.float32),
        jax.ShapeDtypeStruct((E_SUB, MSG_DIM), jnp.float32),
    ],
    mesh=_mesh(),
    scratch_types=[
        pltpu.VMEM((16,), jnp.int32),
        pltpu.VMEM((512,), jnp.int32),
        pltpu.VMEM((512,), jnp.int32),
        pltpu.VMEM((512, MEM_DIM), jnp.float32),
        pltpu.VMEM((256,), jnp.int32),
        pltpu.VMEM((256, MEM_DIM), jnp.float32),
        pltpu.VMEM((512,), jnp.float32),
        pltpu.VMEM((512,), jnp.float32),
        pltpu.SemaphoreType.DMA,
    ],
  )


def _sc_segsum_body(m_hbm, dsub_hbm,
                    agg_out,
                    dst_v, rows_v, acc, sem):
    c = lax.axis_index("c")
    s = lax.axis_index("s")
    del sem
    coff = c * _ACC_OWN

    def _z(i, _):
        rows_v[i // 8, pl.ds((i % 8) * 16, 16)] = jnp.zeros((16,), jnp.float32)
        return 0

    lax.fori_loop(0, 256 * 8, _z, 0)
    pltpu.sync_copy(rows_v, acc.at[pl.ds(s * 544, 256)])
    pltpu.sync_copy(rows_v, acc.at[pl.ds(s * 544 + 256, 256)])
    pltpu.sync_copy(rows_v.at[pl.ds(0, 32)], acc.at[pl.ds(s * 544 + 512, 32)])
    plsc.subcore_barrier()

    for k in range(20):
        eb = s * (E_SUB // _NS) + k * 256
        pltpu.sync_copy(dsub_hbm.at[pl.ds(eb, 256)], dst_v)
        pltpu.sync_copy(m_hbm.at[pl.ds(eb, 256)], rows_v)

        def _rm(j, _):
            d16 = dst_v[pl.ds(j * 16, 16)]
            loc = d16 - coff
            owned = (loc >= 0) & (loc < _ACC_OWN)
            trash = _ACC_OWN + ((lax.iota(jnp.int32, 16) + j * 16) & 511)
            dst_v[pl.ds(j * 16, 16)] = jnp.where(owned, loc, trash)
            return 0

        lax.fori_loop(0, 16, _rm, 0)
        pltpu.sync_copy(rows_v, acc.at[dst_v], add=True)
    plsc.subcore_barrier()
    pltpu.sync_copy(acc.at[pl.ds(s * 512, 512)],
                    agg_out.at[pl.ds(coff + s * 512, 512)])


@functools.cache
def _sc_segsum():
  return pl.kernel(
    _sc_segsum_body,
    out_type=[
        jax.ShapeDtypeStruct((N_SUB, MEM_DIM), jnp.float32),
    ],
    mesh=_mesh(),
    scratch_types=[
        pltpu.VMEM((256,), jnp.int32),
        pltpu.VMEM((256, MEM_DIM), jnp.float32),
        pltpu.VMEM_SHARED((_ACC_ROWS, MEM_DIM), jnp.float32),
        pltpu.SemaphoreType.DMA,
    ],
  )


def _sc_asm_body(t_hbm, tlu_hbm, hout_hbm, gidx_hbm, hidx_hbm,
                 nmem_out, nlu_out, hsd_out,
                 idx256_v, rows256_v, idx_v, rows_v, fa_v, sem):
    c = lax.axis_index("c")
    s = lax.axis_index("s")
    w = c * _NS + s

    pltpu.sync_copy(hidx_hbm.at[pl.ds(w * 256, 256)], idx256_v)
    pltpu.async_copy(hout_hbm.at[idx256_v], rows256_v, sem).wait()
    pltpu.sync_copy(rows256_v, hsd_out.at[pl.ds(w * 256, 256)])

    mb = c * _HALF + s * 3128

    def _rebuild(off, n, ibuf, rbuf, fbuf):
        pltpu.sync_copy(gidx_hbm.at[pl.ds(off, n)], ibuf)
        pltpu.async_copy(t_hbm.at[ibuf], rbuf, sem).wait()
        pltpu.sync_copy(rbuf, nmem_out.at[pl.ds(off, n)])
        pltpu.async_copy(tlu_hbm.at[ibuf], fbuf, sem).wait()
        pltpu.sync_copy(fbuf, nlu_out.at[pl.ds(off, n)])

    for k in range(6):
        _rebuild(mb + k * 512, 512, idx_v, rows_v, fa_v)
    _rebuild(mb + 3072, 8, idx_v.at[pl.ds(0, 8)], rows_v.at[pl.ds(0, 8)],
             fa_v.at[pl.ds(0, 8)])

    @pl.when(s < _NS - 1)
    def _():
        _rebuild(mb + 3080, 48, idx_v.at[pl.ds(0, 48)],
                 rows_v.at[pl.ds(0, 48)], fa_v.at[pl.ds(0, 48)])


@functools.cache
def _sc_asm():
  return pl.kernel(
    _sc_asm_body,
    out_type=[
        jax.ShapeDtypeStruct((NUM_NODES, MEM_DIM), jnp.float32),
        jax.ShapeDtypeStruct((NUM_NODES,), jnp.float32),
        jax.ShapeDtypeStruct((2 * B, MEM_DIM), jnp.float32),
    ],
    mesh=_mesh(),
    scratch_types=[
        pltpu.VMEM((256,), jnp.int32),
        pltpu.VMEM((256, MEM_DIM), jnp.float32),
        pltpu.VMEM((512,), jnp.int32),
        pltpu.VMEM((512, MEM_DIM), jnp.float32),
        pltpu.VMEM((512,), jnp.float32),
        pltpu.SemaphoreType.DMA,
    ],
  )


_BE = 8192


def _edge_msg_body(hfeat_ref, msgf_ref, relt_ref, wn_h_ref, wn_m_ref,
                   wn_t_ref, tw_ref, tb_ref, out_ref):
    f32 = jnp.float32
    enc = jnp.cos(relt_ref[...][:, None] * tw_ref[...][None, :]
                  + tb_ref[...][None, :])
    m = (
        jnp.dot(hfeat_ref[...], wn_h_ref[...], preferred_element_type=f32)
        + jnp.dot(msgf_ref[...], wn_m_ref[...], preferred_element_type=f32)
        + jnp.dot(enc, wn_t_ref[...], preferred_element_type=f32)
    )
    out_ref[...] = m


def _encoder_body(h_ref, agg_ref, wself_ref, b_ref, out_ref):
    f32 = jnp.float32
    hw = jnp.dot(h_ref[...], wself_ref[...], preferred_element_type=f32)
    out_ref[...] = jnp.tanh(hw + agg_ref[...] + b_ref[...][None, :])


def _update_body(mem_src_ref, mem_dst_ref, msg_ref, wu_mem_ref, wu_msg_ref,
                 bu_ref, out_src_ref, out_dst_ref):
    f32 = jnp.float32
    msg_part = jnp.dot(msg_ref[...], wu_msg_ref[...],
                       preferred_element_type=f32)
    out_src_ref[...] = jnp.tanh(
        jnp.dot(mem_src_ref[...], wu_mem_ref[...], preferred_element_type=f32)
        + msg_part + bu_ref[...][None, :])
    out_dst_ref[...] = jnp.tanh(
        jnp.dot(mem_dst_ref[...], wu_mem_ref[...], preferred_element_type=f32)
        + msg_part + bu_ref[...][None, :])


def kernel(edge_index, t, msg, full_msg, full_t, n_id, sub_edge_index, e_id,
           memory, last_update, time_w, time_b, W_self, W_nbr, b_enc, Wu, bu):
    i32 = jnp.int32
    src = edge_index[0].astype(i32)
    dst = edge_index[1].astype(i32)
    src_sub = sub_edge_index[0].astype(i32)
    dst_sub = sub_edge_index[1].astype(i32)
    e_id = e_id.astype(i32)
    n_id = n_id.astype(i32)

    srcdst = jnp.concatenate([src, dst])
    h, mem_sd, rel_t, h_feats, msg_feats = _sc_stage1()(
        memory, last_update, full_t, n_id, src_sub, e_id, full_msg, srcdst)
    mem_src = mem_sd[:B]
    mem_dst = mem_sd[B:]

    wn_h = W_nbr[:MEM_DIM]
    wn_m = W_nbr[MEM_DIM:MEM_DIM + MSG_DIM]
    wn_t = W_nbr[MEM_DIM + MSG_DIM:]
    m = pl.pallas_call(
        _edge_msg_body,
        grid=(E_SUB // _BE,),
        in_specs=[
            pl.BlockSpec((_BE, MEM_DIM), lambda i: (i, 0)),
            pl.BlockSpec((_BE, MSG_DIM), lambda i: (i, 0)),
            pl.BlockSpec((_BE,), lambda i: (i,)),
            pl.BlockSpec((MEM_DIM, MEM_DIM), lambda i: (0, 0)),
            pl.BlockSpec((MSG_DIM, MEM_DIM), lambda i: (0, 0)),
            pl.BlockSpec((TIME_DIM, MEM_DIM), lambda i: (0, 0)),
            pl.BlockSpec((TIME_DIM,), lambda i: (0,)),
            pl.BlockSpec((TIME_DIM,), lambda i: (0,)),
        ],
        out_specs=pl.BlockSpec((_BE, MEM_DIM), lambda i: (i, 0)),
        out_shape=jax.ShapeDtypeStruct((E_SUB, MEM_DIM), jnp.float32),
    )(h_feats, msg_feats, rel_t, wn_h, wn_m, wn_t, time_w, time_b)

    wu_mem = Wu[:MEM_DIM]
    wu_msg = Wu[MEM_DIM:]
    upd_src, upd_dst = pl.pallas_call(
        _update_body,
        grid=(1,),
        in_specs=[
            pl.BlockSpec((B, MEM_DIM), lambda i: (0, 0)),
            pl.BlockSpec((B, MEM_DIM), lambda i: (0, 0)),
            pl.BlockSpec((B, MSG_DIM), lambda i: (0, 0)),
            pl.BlockSpec((MEM_DIM, MEM_DIM), lambda i: (0, 0)),
            pl.BlockSpec((MSG_DIM, MEM_DIM), lambda i: (0, 0)),
            pl.BlockSpec((MEM_DIM,), lambda i: (0,)),
        ],
        out_specs=[
            pl.BlockSpec((B, MEM_DIM), lambda i: (0, 0)),
            pl.BlockSpec((B, MEM_DIM), lambda i: (0, 0)),
        ],
        out_shape=[
            jax.ShapeDtypeStruct((B, MEM_DIM), jnp.float32),
            jax.ShapeDtypeStruct((B, MEM_DIM), jnp.float32),
        ],
    )(mem_src, mem_dst, msg, wu_mem, wu_msg, bu)

    (agg,) = _sc_segsum()(m, dst_sub)

    BN = 4096
    h_out = pl.pallas_call(
        _encoder_body,
        grid=(N_SUB // BN,),
        in_specs=[
            pl.BlockSpec((BN, MEM_DIM), lambda i: (i, 0)),
            pl.BlockSpec((BN, MEM_DIM), lambda i: (i, 0)),
            pl.BlockSpec((MEM_DIM, MEM_DIM), lambda i: (0, 0)),
            pl.BlockSpec((MEM_DIM,), lambda i: (0,)),
        ],
        out_specs=pl.BlockSpec((BN, MEM_DIM), lambda i: (i, 0)),
        out_shape=jax.ShapeDtypeStruct((N_SUB, MEM_DIM), jnp.float32),
    )(h, agg, W_self, b_enc)

    q = jnp.concatenate([src, dst])
    pos = jnp.searchsorted(n_id, q, side="right").astype(i32) - 1
    found = (pos >= 0) & (n_id[jnp.clip(pos, 0)] == q)
    miss_row = N_SUB + (jnp.arange(2 * B, dtype=i32) % 32)
    hidx = jnp.where(found, pos, miss_row).astype(i32)

    rows_all = q
    ordinal = jnp.arange(2 * B, dtype=i32)
    win_slot = jnp.full((NUM_NODES,), -1, i32).at[rows_all].max(ordinal)
    tval_all = jnp.concatenate([t, t])
    upd_cat = jnp.concatenate([upd_src, upd_dst], axis=0)
    t_table = jnp.concatenate([upd_cat, memory], axis=0)
    tlu_table = jnp.concatenate([tval_all, last_update])
    gidx_full = jnp.where(
        win_slot >= 0, win_slot,
        jnp.arange(NUM_NODES, dtype=i32) + 2 * B).astype(i32)

    h_ext = jnp.concatenate(
        [h_out, jnp.broadcast_to(h_out[0:1], (32, MEM_DIM))], axis=0)
    new_memory, new_last_update, hsd = _sc_asm()(
        t_table, tlu_table, h_ext, gidx_full, hidx)
    h_src = hsd[:B]
    h_dst = hsd[B:]

    return (h_src, h_dst, new_memory, new_last_update)
